# Initial kernel scaffold; baseline (speedup 1.0000x reference)
import jax
import jax.numpy as jnp
from jax.experimental import pallas as pl


def kernel(x, w1, g1, be1, m1, v1, w2, g2, be2, m2, v2):
    raise NotImplementedError("write your pallas kernel here")


# speedup vs baseline: 1.2669x; 1.2669x over previous
import functools

import jax
import jax.numpy as jnp
from jax.experimental import pallas as pl
from jax.experimental.pallas import tpu as pltpu

_LANE = 128
_TH = 28


def _block_body(xp_ref, w1_ref, b1_ref, w2_ref, b2_ref, o_ref, *, th, w, h):
    c = _LANE
    ht = pl.program_id(1)
    r0 = pl.multiple_of(ht * th, th)

    xs = xp_ref[pl.ds(r0, th + 4)]

    m1 = (th + 2) * w
    patch1 = jnp.concatenate(
        [xs[ky:ky + th + 2, kx:kx + w, :] for ky in range(3) for kx in range(3)],
        axis=-1).reshape(m1, 9 * c)
    acc1 = jnp.dot(patch1, w1_ref[...], preferred_element_type=jnp.float32)
    out1 = jnp.maximum(acc1 + b1_ref[0], 0.0)
    out1 = out1.reshape(th + 2, w, c).astype(jnp.bfloat16)

    g = jax.lax.broadcasted_iota(jnp.int32, (th + 2, 1, 1), 0) + (r0 - 1)
    out1 = jnp.where((g >= 0) & (g < h), out1, jnp.bfloat16(0))

    zc = jnp.zeros((th + 2, 1, c), jnp.bfloat16)
    mid = jnp.concatenate([zc, out1, zc], axis=1)

    m2 = th * w
    patch2 = jnp.concatenate(
        [mid[ky:ky + th, kx:kx + w, :] for ky in range(3) for kx in range(3)],
        axis=-1).reshape(m2, 9 * c)
    acc2 = jnp.dot(patch2, w2_ref[...], preferred_element_type=jnp.float32)

    ident = xs[2:th + 2, 1:w + 1, :].reshape(m2, c).astype(jnp.float32)
    res = jnp.maximum(acc2 + b2_ref[0] + ident, 0.0)
    o_ref[...] = res.reshape(th, w, c)


def _basic_block(x_nchw, w1, g1, be1, m1, v1, w2, g2, be2, m2, v2, eps=1e-5):
    B, C, H, W = x_nchw.shape
    assert C == _LANE and w1.shape[0] == C
    th = _TH if H % _TH == 0 else H
    nt = H // th
    wp = ((W + 2 + 7) // 8) * 8

    s1 = g1 / jnp.sqrt(v1 + eps)
    s2 = g2 / jnp.sqrt(v2 + eps)
    b1 = (be1 - m1 * s1).reshape(1, C).astype(jnp.float32)
    b2 = (be2 - m2 * s2).reshape(1, C).astype(jnp.float32)

    def prep_w(wt, s):
        whwio = jnp.transpose(wt, (2, 3, 1, 0)) * s[None, None, None, :]
        return whwio.reshape(9 * C, C).astype(jnp.bfloat16)

    w1f = prep_w(w1, s1)
    w2f = prep_w(w2, s2)

    x_nhwc = jnp.transpose(x_nchw, (0, 2, 3, 1)).astype(jnp.bfloat16)
    xp = jnp.pad(x_nhwc, ((0, 0), (2, 2), (1, wp - W - 1), (0, 0)))

    body = functools.partial(_block_body, th=th, w=W, h=H)
    out_nhwc = pl.pallas_call(
        body,
        out_shape=jax.ShapeDtypeStruct((B, H, W, C), jnp.float32),
        grid_spec=pltpu.PrefetchScalarGridSpec(
            num_scalar_prefetch=0,
            grid=(B, nt),
            in_specs=[
                pl.BlockSpec((None, H + 4, wp, C), lambda b, t: (b, 0, 0, 0)),
                pl.BlockSpec((9 * C, C), lambda b, t: (0, 0)),
                pl.BlockSpec((1, C), lambda b, t: (0, 0)),
                pl.BlockSpec((9 * C, C), lambda b, t: (0, 0)),
                pl.BlockSpec((1, C), lambda b, t: (0, 0)),
            ],
            out_specs=pl.BlockSpec((None, th, W, C), lambda b, t: (b, t, 0, 0)),
        ),
        compiler_params=pltpu.CompilerParams(
            dimension_semantics=("parallel", "parallel")),
    )(xp, w1f, b1, w2f, b2)

    return jnp.transpose(out_nhwc, (0, 3, 1, 2))


def kernel(x, w1, g1, be1, m1, v1, w2, g2, be2, m2, v2):
    return _basic_block(x, w1, g1, be1, m1, v1, w2, g2, be2, m2, v2)


# speedup vs baseline: 1.2939x; 1.0213x over previous
import jax
import jax.numpy as jnp
from jax.experimental import pallas as pl
from jax.experimental.pallas import tpu as pltpu

_LANE = 128
_SH = 28


def _subtile(xs, w1, b1, w2, b2, r0, sh, w, h):
    c = _LANE

    m1 = (sh + 2) * w
    patch1 = jnp.concatenate(
        [xs[ky:ky + sh + 2, kx:kx + w, :] for ky in range(3) for kx in range(3)],
        axis=-1).reshape(m1, 9 * c)
    acc1 = jnp.dot(patch1, w1, preferred_element_type=jnp.float32)
    out1 = jnp.maximum(acc1 + b1, 0.0)
    out1 = out1.reshape(sh + 2, w, c).astype(jnp.bfloat16)

    g = jax.lax.broadcasted_iota(jnp.int32, (sh + 2, 1, 1), 0) + (r0 - 1)
    out1 = jnp.where((g >= 0) & (g < h), out1, jnp.bfloat16(0))

    zc = jnp.zeros((sh + 2, 1, c), jnp.bfloat16)
    mid = jnp.concatenate([zc, out1, zc], axis=1)

    m2 = sh * w
    patch2 = jnp.concatenate(
        [mid[ky:ky + sh, kx:kx + w, :] for ky in range(3) for kx in range(3)],
        axis=-1).reshape(m2, 9 * c)
    acc2 = jnp.dot(patch2, w2, preferred_element_type=jnp.float32)

    ident = xs[2:sh + 2, 1:w + 1, :].reshape(m2, c).astype(jnp.float32)
    return jnp.maximum(acc2 + b2 + ident, 0.0)


def _block_body(xp_ref, w1_ref, b1_ref, w2_ref, b2_ref, o_ref, *, sh, w, h):
    w1 = w1_ref[...]
    b1 = b1_ref[0]
    w2 = w2_ref[...]
    b2 = b2_ref[0]
    results = []
    for t in range(h // sh):
        xs = xp_ref[t * sh:t * sh + sh + 4]
        results.append(_subtile(xs, w1, b1, w2, b2, t * sh, sh, w, h))
    for t, res in enumerate(results):
        o_ref[t * sh:(t + 1) * sh] = res.reshape(sh, w, _LANE)


def _basic_block(x_nchw, w1, g1, be1, m1, v1, w2, g2, be2, m2, v2, eps=1e-5):
    B, C, H, W = x_nchw.shape
    assert C == _LANE and w1.shape[0] == C
    sh = _SH if H % _SH == 0 else H
    wp = ((W + 2 + 7) // 8) * 8

    s1 = g1 / jnp.sqrt(v1 + eps)
    s2 = g2 / jnp.sqrt(v2 + eps)
    b1 = (be1 - m1 * s1).reshape(1, C).astype(jnp.float32)
    b2 = (be2 - m2 * s2).reshape(1, C).astype(jnp.float32)

    def prep_w(wt, s):
        whwio = jnp.transpose(wt, (2, 3, 1, 0)) * s[None, None, None, :]
        return whwio.reshape(9 * C, C).astype(jnp.bfloat16)

    w1f = prep_w(w1, s1)
    w2f = prep_w(w2, s2)

    x_nhwc = jnp.transpose(x_nchw, (0, 2, 3, 1)).astype(jnp.bfloat16)
    xp = jnp.pad(x_nhwc, ((0, 0), (2, 2), (1, wp - W - 1), (0, 0)))

    def body(*refs):
        return _block_body(*refs, sh=sh, w=W, h=H)

    out_nhwc = pl.pallas_call(
        body,
        out_shape=jax.ShapeDtypeStruct((B, H, W, C), jnp.float32),
        grid_spec=pltpu.PrefetchScalarGridSpec(
            num_scalar_prefetch=0,
            grid=(B,),
            in_specs=[
                pl.BlockSpec((None, H + 4, wp, C), lambda b: (b, 0, 0, 0)),
                pl.BlockSpec((9 * C, C), lambda b: (0, 0)),
                pl.BlockSpec((1, C), lambda b: (0, 0)),
                pl.BlockSpec((9 * C, C), lambda b: (0, 0)),
                pl.BlockSpec((1, C), lambda b: (0, 0)),
            ],
            out_specs=pl.BlockSpec((None, H, W, C), lambda b: (b, 0, 0, 0)),
        ),
        compiler_params=pltpu.CompilerParams(
            dimension_semantics=("parallel",)),
    )(xp, w1f, b1, w2f, b2)

    return jnp.transpose(out_nhwc, (0, 3, 1, 2))


def kernel(x, w1, g1, be1, m1, v1, w2, g2, be2, m2, v2):
    return _basic_block(x, w1, g1, be1, m1, v1, w2, g2, be2, m2, v2)


# speedup vs baseline: 2.2348x; 1.7272x over previous
import jax
import jax.numpy as jnp
from jax.experimental import pallas as pl
from jax.experimental.pallas import tpu as pltpu

_LANE = 128
_SH = 28


def _conv3x3(src_f, rhs01, rhs2, m, w):
    c = _LANE
    p01 = jnp.dot(src_f[0:m + w], rhs01, preferred_element_type=jnp.float32)
    p2 = jnp.dot(src_f[2 * w:2 * w + m], rhs2,
                 preferred_element_type=jnp.float32)
    return p01[0:m, 0:c] + p01[w:m + w, c:2 * c] + p2


def _subtile(xs, rhs1a, rhs1b, b1, rhs2a, rhs2b, b2, r0, sh, w, h):
    c = _LANE

    xw3 = jnp.concatenate([xs[:, kx:kx + w, :] for kx in range(3)],
                          axis=-1).reshape((sh + 4) * w, 3 * c)
    m1 = (sh + 2) * w
    out1 = jnp.maximum(_conv3x3(xw3, rhs1a, rhs1b, m1, w) + b1, 0.0)
    out1 = out1.reshape(sh + 2, w, c).astype(jnp.bfloat16)

    g = jax.lax.broadcasted_iota(jnp.int32, (sh + 2, 1, 1), 0) + (r0 - 1)
    out1 = jnp.where((g >= 0) & (g < h), out1, jnp.bfloat16(0))

    zc = jnp.zeros((sh + 2, 1, c), jnp.bfloat16)
    pk0 = jnp.concatenate([zc, out1[:, 0:w - 1, :]], axis=1)
    pk2 = jnp.concatenate([out1[:, 1:w, :], zc], axis=1)
    mw3 = jnp.concatenate([pk0, out1, pk2], axis=-1).reshape((sh + 2) * w,
                                                            3 * c)
    m2 = sh * w
    acc2 = _conv3x3(mw3, rhs2a, rhs2b, m2, w)

    ident = xs[2:sh + 2, 1:w + 1, :].reshape(m2, c).astype(jnp.float32)
    return jnp.maximum(acc2 + b2 + ident, 0.0)


def _block_body(xp_ref, r1a_ref, r1b_ref, b1_ref, r2a_ref, r2b_ref, b2_ref,
                o_ref, *, sh, w, h):
    ws = (r1a_ref[...], r1b_ref[...], b1_ref[0], r2a_ref[...], r2b_ref[...],
          b2_ref[0])
    results = []
    for t in range(h // sh):
        xs = xp_ref[t * sh:t * sh + sh + 4]
        results.append(_subtile(xs, *ws, t * sh, sh, w, h))
    for t, res in enumerate(results):
        o_ref[t * sh:(t + 1) * sh] = res.reshape(sh, w, _LANE)


def _basic_block(x_nchw, w1, g1, be1, m1, v1, w2, g2, be2, m2, v2, eps=1e-5):
    B, C, H, W = x_nchw.shape
    assert C == _LANE and w1.shape[0] == C
    sh = _SH if H % _SH == 0 else H
    wp = ((W + 2 + 7) // 8) * 8

    s1 = g1 / jnp.sqrt(v1 + eps)
    s2 = g2 / jnp.sqrt(v2 + eps)
    b1 = (be1 - m1 * s1).reshape(1, C).astype(jnp.float32)
    b2 = (be2 - m2 * s2).reshape(1, C).astype(jnp.float32)

    def prep_w(wt, s):
        whwio = jnp.transpose(wt, (2, 3, 1, 0)) * s[None, None, None, :]
        per_ky = whwio.reshape(3, 3 * C, C).astype(jnp.bfloat16)
        return (jnp.concatenate([per_ky[0], per_ky[1]], axis=1), per_ky[2])

    w1a, w1b = prep_w(w1, s1)
    w2a, w2b = prep_w(w2, s2)

    x_nhwc = jnp.transpose(x_nchw, (0, 2, 3, 1)).astype(jnp.bfloat16)
    xp = jnp.pad(x_nhwc, ((0, 0), (2, 2), (1, wp - W - 1), (0, 0)))

    def body(*refs):
        return _block_body(*refs, sh=sh, w=W, h=H)

    out_nhwc = pl.pallas_call(
        body,
        out_shape=jax.ShapeDtypeStruct((B, H, W, C), jnp.float32),
        grid_spec=pltpu.PrefetchScalarGridSpec(
            num_scalar_prefetch=0,
            grid=(B,),
            in_specs=[
                pl.BlockSpec((None, H + 4, wp, C), lambda b: (b, 0, 0, 0)),
                pl.BlockSpec((3 * C, 2 * C), lambda b: (0, 0)),
                pl.BlockSpec((3 * C, C), lambda b: (0, 0)),
                pl.BlockSpec((1, C), lambda b: (0, 0)),
                pl.BlockSpec((3 * C, 2 * C), lambda b: (0, 0)),
                pl.BlockSpec((3 * C, C), lambda b: (0, 0)),
                pl.BlockSpec((1, C), lambda b: (0, 0)),
            ],
            out_specs=pl.BlockSpec((None, H, W, C), lambda b: (b, 0, 0, 0)),
        ),
        compiler_params=pltpu.CompilerParams(
            dimension_semantics=("parallel",)),
    )(xp, w1a, w1b, b1, w2a, w2b, b2)

    return jnp.transpose(out_nhwc, (0, 3, 1, 2))


def kernel(x, w1, g1, be1, m1, v1, w2, g2, be2, m2, v2):
    return _basic_block(x, w1, g1, be1, m1, v1, w2, g2, be2, m2, v2)
